# Initial kernel scaffold; baseline (speedup 1.0000x reference)
import jax
import jax.numpy as jnp
from jax.experimental import pallas as pl


def kernel(prototype_activations, upsampled_activation, W):
    raise NotImplementedError("write your pallas kernel here")


# speedup vs baseline: 9.9767x; 9.9767x over previous
import functools

import jax
import jax.numpy as jnp
from jax import lax
from jax.experimental import pallas as pl
from jax.experimental.pallas import tpu as pltpu
from jax.experimental.pallas import tpu_sc as plsc

_LANES = 16
_NUM_CORES = 2
_NUM_SUBCORES = 16
_NUM_WORKERS = _NUM_CORES * _NUM_SUBCORES


def _row_max(buf, start, hw):
    vals = [buf[pl.ds(start + k * _LANES, _LANES)] for k in range(hw // _LANES)]
    while len(vals) > 1:
        nxt = [jnp.maximum(vals[i], vals[i + 1]) for i in range(0, len(vals) - 1, 2)]
        if len(vals) % 2:
            nxt.append(vals[-1])
        vals = nxt
    return vals[0]


@functools.cache
def _make_sc_pool(num_rows, hw, chunk):
    rows_per_w = num_rows // _NUM_WORKERS
    n_chunks = rows_per_w // chunk
    mesh = plsc.VectorSubcoreMesh(core_axis_name="c", subcore_axis_name="s")

    stride = _LANES + 1

    @functools.partial(
        pl.kernel,
        mesh=mesh,
        out_type=jax.ShapeDtypeStruct((num_rows,), jnp.float32),
        compiler_params=pltpu.CompilerParams(needs_layout_passes=False),
        scratch_types=[
            pltpu.VMEM((chunk * hw,), jnp.float32),
            pltpu.VMEM((chunk * hw,), jnp.float32),
            pltpu.VMEM((rows_per_w,), jnp.float32),
            pltpu.VMEM((_LANES * stride,), jnp.float32),
            pltpu.SemaphoreType.DMA,
            pltpu.SemaphoreType.DMA,
        ],
    )
    def sc_pool(acts_hbm, out_hbm, buf0, buf1, out_v, accs, sem0, sem1):
        wid = lax.axis_index("s") * _NUM_CORES + lax.axis_index("c")
        row0 = wid * rows_per_w
        bufs = (buf0, buf1)
        sems = (sem0, sem1)
        lane = lax.iota(jnp.int32, _LANES)
        col0 = lane * stride

        def start_dma(g):
            return pltpu.async_copy(
                acts_hbm.at[pl.ds((row0 + g * chunk) * hw, chunk * hw)],
                bufs[g % 2], sems[g % 2])

        descs = [None, None]
        descs[0] = start_dma(0)
        for g in range(n_chunks):
            if g + 1 < n_chunks:
                descs[(g + 1) % 2] = start_dma(g + 1)
            descs[g % 2].wait()
            buf = bufs[g % 2]
            out_base = g * chunk

            def group_body(q, carry, buf=buf, out_base=out_base):
                def row_body(i, carry2, buf=buf, q=q):
                    m = _row_max(buf, (q * _LANES + i) * hw, hw)
                    accs[pl.ds(i * stride, _LANES)] = m
                    return carry2

                lax.fori_loop(0, _LANES, row_body, 0)
                res = plsc.load_gather(accs, [col0])
                for j in range(1, _LANES):
                    res = jnp.maximum(res, plsc.load_gather(accs, [col0 + j]))
                out_v[pl.ds(out_base + q * _LANES, _LANES)] = res
                return carry

            lax.fori_loop(0, chunk // _LANES, group_body, 0)

        pltpu.sync_copy(out_v, out_hbm.at[pl.ds(row0, rows_per_w)])

    return sc_pool


def _mm_body(sim_ref, w_ref, o_ref):
    o_ref[...] = lax.dot_general(
        sim_ref[...], w_ref[...], (((1,), (1,)), ((), ())),
        preferred_element_type=jnp.float32)


def kernel(prototype_activations, upsampled_activation, W):
    B, P, H, Wsp = prototype_activations.shape
    hw = H * Wsp
    num_rows = B * P
    acts_flat = prototype_activations.reshape(num_rows * hw)
    sim_flat = _make_sc_pool(num_rows, hw, 80)(acts_flat)
    sim = sim_flat.reshape(B, P)
    C = W.shape[0]
    logits = pl.pallas_call(
        _mm_body,
        out_shape=jax.ShapeDtypeStruct((B, C), jnp.float32),
    )(sim, W)
    return logits


# speedup vs baseline: 95.9274x; 9.6151x over previous
import functools

import jax
import jax.numpy as jnp
from jax import lax
from jax.experimental import pallas as pl
from jax.experimental.pallas import tpu as pltpu
from jax.experimental.pallas import tpu_sc as plsc

_LANES = 16
_NUM_CORES = 2
_NUM_SUBCORES = 16
_NUM_WORKERS = _NUM_CORES * _NUM_SUBCORES
_NEG_INF = float("-inf")


@functools.cache
def _make_sc_pool(num_slabs, rows, p):
    chunk = _LANES
    n_chunks = rows // chunk
    n_pv = p // _LANES
    mesh = plsc.VectorSubcoreMesh(core_axis_name="c", subcore_axis_name="s")

    @functools.partial(
        pl.kernel,
        mesh=mesh,
        out_type=jax.ShapeDtypeStruct((num_slabs * p,), jnp.float32),
        compiler_params=pltpu.CompilerParams(needs_layout_passes=False),
        scratch_types=[
            pltpu.VMEM((chunk, p), jnp.float32),
            pltpu.VMEM((chunk, p), jnp.float32),
            pltpu.VMEM((p,), jnp.float32),
            pltpu.SemaphoreType.DMA,
            pltpu.SemaphoreType.DMA,
        ],
    )
    def sc_pool(acts_hbm, out_hbm, buf0, buf1, acc_v, sem0, sem1):
        wid = lax.axis_index("s") * _NUM_CORES + lax.axis_index("c")
        row0 = wid * rows
        bufs = (buf0, buf1)
        sems = (sem0, sem1)

        def start_dma(g):
            return pltpu.async_copy(
                acts_hbm.at[pl.ds(row0 + g * chunk, chunk), :],
                bufs[g % 2], sems[g % 2])

        descs = [None, None]
        descs[0] = start_dma(0)
        for g in range(n_chunks):
            if g + 1 < n_chunks:
                descs[(g + 1) % 2] = start_dma(g + 1)
            descs[g % 2].wait()
            buf = bufs[g % 2]

            if g == 0:
                def col_init(pv, carry, buf=buf):
                    c0 = pv * _LANES
                    m = buf[0, pl.ds(c0, _LANES)]
                    for s in range(1, chunk):
                        m = jnp.maximum(m, buf[s, pl.ds(c0, _LANES)])
                    acc_v[pl.ds(c0, _LANES)] = m
                    return carry

                lax.fori_loop(0, n_pv, col_init, 0)
            else:
                def col_body(pv, carry, buf=buf):
                    c0 = pv * _LANES
                    m = acc_v[pl.ds(c0, _LANES)]
                    for s in range(chunk):
                        m = jnp.maximum(m, buf[s, pl.ds(c0, _LANES)])
                    acc_v[pl.ds(c0, _LANES)] = m
                    return carry

                lax.fori_loop(0, n_pv, col_body, 0)

        pltpu.sync_copy(acc_v, out_hbm.at[pl.ds(wid * p, p)])

    return sc_pool


def _mm_body(sim_ref, w_ref, o_ref):
    o_ref[...] = lax.dot_general(
        sim_ref[...], w_ref[...], (((1,), (1,)), ((), ())),
        preferred_element_type=jnp.float32)


def kernel(prototype_activations, upsampled_activation, W):
    B, P, H, Wsp = prototype_activations.shape
    hw = H * Wsp
    xt = prototype_activations.transpose(0, 2, 3, 1).reshape(B * hw, P)
    sim_flat = _make_sc_pool(B, hw, P)(xt)
    sim = sim_flat.reshape(B, P)
    C = W.shape[0]
    logits = pl.pallas_call(
        _mm_body,
        out_shape=jax.ShapeDtypeStruct((B, C), jnp.float32),
    )(sim, W)
    return logits


# speedup vs baseline: 101.3868x; 1.0569x over previous
import functools

import jax
import jax.numpy as jnp
from jax import lax
from jax.experimental import pallas as pl
from jax.experimental.pallas import tpu as pltpu
from jax.experimental.pallas import tpu_sc as plsc

_LANES = 16
_NUM_CORES = 2
_NUM_SUBCORES = 16
_NUM_WORKERS = _NUM_CORES * _NUM_SUBCORES
_NEG_INF = float("-inf")


def _tree_max(vals):
    while len(vals) > 1:
        nxt = [jnp.maximum(vals[i], vals[i + 1]) for i in range(0, len(vals) - 1, 2)]
        if len(vals) % 2:
            nxt.append(vals[-1])
        vals = nxt
    return vals[0]


@functools.cache
def _make_sc_pool(num_slabs, rows, p):
    chunk = 24
    n_chunks = rows // chunk
    pv_unroll = 5
    n_pv_iter = p // (_LANES * pv_unroll)
    mesh = plsc.VectorSubcoreMesh(core_axis_name="c", subcore_axis_name="s")

    @functools.partial(
        pl.kernel,
        mesh=mesh,
        out_type=jax.ShapeDtypeStruct((num_slabs * p,), jnp.float32),
        compiler_params=pltpu.CompilerParams(needs_layout_passes=False),
        scratch_types=[
            pltpu.VMEM((chunk, p), jnp.float32),
            pltpu.VMEM((chunk, p), jnp.float32),
            pltpu.VMEM((p,), jnp.float32),
            pltpu.SemaphoreType.DMA,
            pltpu.SemaphoreType.DMA,
        ],
    )
    def sc_pool(acts_hbm, out_hbm, buf0, buf1, acc_v, sem0, sem1):
        wid = lax.axis_index("s") * _NUM_CORES + lax.axis_index("c")
        row0 = wid * rows
        bufs = (buf0, buf1)
        sems = (sem0, sem1)
        ninf = jnp.full((_LANES,), _NEG_INF, jnp.float32)

        @pl.loop(0, p // _LANES)
        def _init(pv):
            acc_v[pl.ds(pv * _LANES, _LANES)] = ninf

        for b in range(2):
            pltpu.async_copy(
                acts_hbm.at[pl.ds(row0 + b * chunk, chunk), :], bufs[b], sems[b])

        @pl.loop(0, n_chunks, step=2)
        def _chunks(g):
            for b in range(2):
                gi = g + b
                pltpu.make_async_copy(
                    acts_hbm.at[pl.ds(0, chunk), :], bufs[b], sems[b]).wait()

                @pl.loop(0, n_pv_iter)
                def _cols(i, b=b):
                    base = i * (_LANES * pv_unroll)
                    for u in range(pv_unroll):
                        c0 = base + u * _LANES
                        vals = [bufs[b][s, pl.ds(c0, _LANES)]
                                for s in range(chunk)]
                        vals.append(acc_v[pl.ds(c0, _LANES)])
                        acc_v[pl.ds(c0, _LANES)] = _tree_max(vals)

                @pl.when(gi + 2 < n_chunks)
                def _refill(b=b, gi=gi):
                    pltpu.async_copy(
                        acts_hbm.at[pl.ds(row0 + (gi + 2) * chunk, chunk), :],
                        bufs[b], sems[b])

        pltpu.sync_copy(acc_v, out_hbm.at[pl.ds(wid * p, p)])

    return sc_pool


def _mm_body(sim_ref, w_ref, o_ref):
    o_ref[...] = lax.dot_general(
        sim_ref[...], w_ref[...], (((1,), (1,)), ((), ())),
        preferred_element_type=jnp.float32)


def kernel(prototype_activations, upsampled_activation, W):
    B, P, H, Wsp = prototype_activations.shape
    hw = H * Wsp
    xt = prototype_activations.transpose(0, 2, 3, 1).reshape(B * hw, P)
    sim_flat = _make_sc_pool(B, hw, P)(xt)
    sim = sim_flat.reshape(B, P)
    C = W.shape[0]
    logits = pl.pallas_call(
        _mm_body,
        out_shape=jax.ShapeDtypeStruct((B, C), jnp.float32),
    )(sim, W)
    return logits


# speedup vs baseline: 123.4652x; 1.2178x over previous
import functools

import jax
import jax.numpy as jnp
from jax import lax
from jax.experimental import pallas as pl
from jax.experimental.pallas import tpu as pltpu
from jax.experimental.pallas import tpu_sc as plsc

_LANES = 16
_NUM_CORES = 2
_NUM_SUBCORES = 16
_NUM_WORKERS = _NUM_CORES * _NUM_SUBCORES
_NEG_INF = float("-inf")


def _tree_max(vals):
    while len(vals) > 1:
        nxt = [jnp.maximum(vals[i], vals[i + 1]) for i in range(0, len(vals) - 1, 2)]
        if len(vals) % 2:
            nxt.append(vals[-1])
        vals = nxt
    return vals[0]


@functools.cache
def _make_sc_pool(num_slabs, rows, slab_stride, p):
    chunk = 24
    n_chunks = rows // chunk
    pv_unroll = 5
    n_pv_iter = p // (_LANES * pv_unroll)
    mesh = plsc.VectorSubcoreMesh(core_axis_name="c", subcore_axis_name="s")

    @functools.partial(
        pl.kernel,
        mesh=mesh,
        out_type=jax.ShapeDtypeStruct((num_slabs * p,), jnp.float32),
        compiler_params=pltpu.CompilerParams(needs_layout_passes=False),
        scratch_types=[
            pltpu.VMEM((chunk, p), jnp.float32),
            pltpu.VMEM((chunk, p), jnp.float32),
            pltpu.VMEM((p,), jnp.float32),
            pltpu.SemaphoreType.DMA,
            pltpu.SemaphoreType.DMA,
        ],
    )
    def sc_pool(acts_hbm, out_hbm, buf0, buf1, acc_v, sem0, sem1):
        wid = lax.axis_index("s") * _NUM_CORES + lax.axis_index("c")
        row0 = wid * slab_stride
        bufs = (buf0, buf1)
        sems = (sem0, sem1)
        ninf = jnp.full((_LANES,), _NEG_INF, jnp.float32)

        @pl.loop(0, p // _LANES)
        def _init(pv):
            acc_v[pl.ds(pv * _LANES, _LANES)] = ninf

        for b in range(2):
            pltpu.async_copy(
                acts_hbm.at[pl.ds(row0 + b * chunk, chunk), :], bufs[b], sems[b])

        @pl.loop(0, n_chunks, step=2)
        def _chunks(g):
            for b in range(2):
                gi = g + b
                pltpu.make_async_copy(
                    acts_hbm.at[pl.ds(0, chunk), :], bufs[b], sems[b]).wait()

                @pl.loop(0, n_pv_iter)
                def _cols(i, b=b):
                    base = i * (_LANES * pv_unroll)
                    for u in range(pv_unroll):
                        c0 = base + u * _LANES
                        vals = [bufs[b][s, pl.ds(c0, _LANES)]
                                for s in range(chunk)]
                        vals.append(acc_v[pl.ds(c0, _LANES)])
                        acc_v[pl.ds(c0, _LANES)] = _tree_max(vals)

                @pl.when(gi + 2 < n_chunks)
                def _refill(b=b, gi=gi):
                    pltpu.async_copy(
                        acts_hbm.at[pl.ds(row0 + (gi + 2) * chunk, chunk), :],
                        bufs[b], sems[b])

        pltpu.sync_copy(acc_v, out_hbm.at[pl.ds(wid * p, p)])

    return sc_pool


def _tc_pool_body(x_ref, o_ref):
    k = pl.program_id(0)
    m = jnp.max(x_ref[...], axis=1)

    @pl.when(k == 0)
    def _init():
        o_ref[...] = m

    @pl.when(k > 0)
    def _fold():
        o_ref[...] = jnp.maximum(o_ref[...], m)


def _mm_body(sc_ref, tc_ref, w_ref, o_ref):
    sim = jnp.maximum(sc_ref[...], tc_ref[...])
    o_ref[...] = lax.dot_general(
        sim, w_ref[...], (((1,), (1,)), ((), ())),
        preferred_element_type=jnp.float32)


_SC_ROWS = 336
_TC_BLK = 48


def kernel(prototype_activations, upsampled_activation, W):
    B, P, H, Wsp = prototype_activations.shape
    hw = H * Wsp
    C = W.shape[0]
    xt = prototype_activations.transpose(0, 2, 3, 1).reshape(B * hw, P)
    sc_flat = _make_sc_pool(B, _SC_ROWS, hw, P)(xt)
    sc_part = sc_flat.reshape(B, P)
    xt3 = xt.reshape(B, hw, P)
    n_tc_blocks = (hw - _SC_ROWS) // _TC_BLK
    tc_part = pl.pallas_call(
        _tc_pool_body,
        grid=(n_tc_blocks,),
        in_specs=[pl.BlockSpec((B, _TC_BLK, P),
                               lambda k: (0, _SC_ROWS // _TC_BLK + k, 0))],
        out_specs=pl.BlockSpec((B, P), lambda k: (0, 0)),
        out_shape=jax.ShapeDtypeStruct((B, P), jnp.float32),
    )(xt3)
    logits = pl.pallas_call(
        _mm_body,
        out_shape=jax.ShapeDtypeStruct((B, C), jnp.float32),
    )(sc_part, tc_part, W)
    return logits
